# Initial kernel scaffold; baseline (speedup 1.0000x reference)
import jax
import jax.numpy as jnp
from jax.experimental import pallas as pl


def kernel(x, edge_index, in_W, in_b, lin_l_W, lin_l_b, lin_r_W, ln_scale, ln_bias, out_W, out_b):
    raise NotImplementedError("write your pallas kernel here")


# speedup vs baseline: 3.8018x; 3.8018x over previous
import functools

import jax
import jax.numpy as jnp
from jax import lax
from jax.experimental import pallas as pl
from jax.experimental.pallas import tpu as pltpu
from jax.experimental.pallas import tpu_sc as plsc

N = 10000
E = 320000
D = 128
L = 3
OUT = 1
EPS = 1e-5

NC = 2
NS = 16
NW = NC * NS
CHUNK = 128
N_PAD = 10112
ROWS_PER_TILE = N_PAD // NS
E_PAD = ((E + NW * CHUNK - 1) // (NW * CHUNK)) * (NW * CHUNK)
EPW = E_PAD // NW
CHUNKS_PER_W = EPW // CHUNK

R = 1000
GRID = N // R


def _sc_agg_body(hn, srcp, dstp, zrows, parts, sidx, didx, rows, sem, acc):
    c = lax.axis_index("c")
    s = lax.axis_index("s")
    wid = s * NC + c
    row0 = s * ROWS_PER_TILE

    pltpu.sync_copy(zrows, acc.at[pl.ds(row0, ROWS_PER_TILE)])
    plsc.subcore_barrier()

    base0 = wid * EPW

    @pl.loop(0, CHUNKS_PER_W)
    def _(g):
        base = pl.multiple_of(base0 + g * CHUNK, CHUNK)
        pltpu.sync_copy(srcp.at[pl.ds(base, CHUNK)], sidx)
        pltpu.sync_copy(dstp.at[pl.ds(base, CHUNK)], didx)
        pltpu.async_copy(hn.at[sidx], rows, sem).wait()
        pltpu.sync_copy(rows, acc.at[didx], add=True)

    plsc.subcore_barrier()
    pltpu.sync_copy(acc.at[pl.ds(row0, ROWS_PER_TILE)],
                    parts.at[c, pl.ds(row0, ROWS_PER_TILE)])


def _sc_deg_body(dstp, zrows, ones, degparts, didx, onesv, acc):
    c = lax.axis_index("c")
    s = lax.axis_index("s")
    wid = s * NC + c
    row0 = s * ROWS_PER_TILE

    pltpu.sync_copy(zrows, acc.at[pl.ds(row0, ROWS_PER_TILE)])
    pltpu.sync_copy(ones, onesv)
    plsc.subcore_barrier()

    base0 = wid * EPW

    @pl.loop(0, CHUNKS_PER_W)
    def _(g):
        base = pl.multiple_of(base0 + g * CHUNK, CHUNK)
        pltpu.sync_copy(dstp.at[pl.ds(base, CHUNK)], didx)
        pltpu.sync_copy(onesv, acc.at[didx], add=True)

    plsc.subcore_barrier()
    pltpu.sync_copy(acc.at[pl.ds(row0, ROWS_PER_TILE)],
                    degparts.at[c, pl.ds(row0, ROWS_PER_TILE)])


def _mesh():
    return plsc.VectorSubcoreMesh(core_axis_name="c", subcore_axis_name="s",
                                  num_cores=NC, num_subcores=NS)


@functools.cache
def _make_sc_agg():
    return pl.kernel(
        _sc_agg_body,
        out_type=jax.ShapeDtypeStruct((NC, N_PAD, D), jnp.float32),
        mesh=_mesh(),
        scratch_types=(
            pltpu.VMEM((CHUNK,), jnp.int32),
            pltpu.VMEM((CHUNK,), jnp.int32),
            pltpu.VMEM((CHUNK, D), jnp.float32),
            pltpu.SemaphoreType.DMA,
            pltpu.VMEM_SHARED((N_PAD, D), jnp.float32),
        ),
    )


@functools.cache
def _make_sc_deg():
    return pl.kernel(
        _sc_deg_body,
        out_type=jax.ShapeDtypeStruct((NC, N_PAD, D), jnp.float32),
        mesh=_mesh(),
        scratch_types=(
            pltpu.VMEM((CHUNK,), jnp.int32),
            pltpu.VMEM((CHUNK, D), jnp.float32),
            pltpu.VMEM_SHARED((N_PAD, D), jnp.float32),
        ),
    )


def _ln(h, scale, bias):
    mu = jnp.mean(h, axis=1, keepdims=True)
    d = h - mu
    var = jnp.mean(d * d, axis=1, keepdims=True)
    return d * lax.rsqrt(var + EPS) * scale + bias


def _tc_in_body(x, wt, b, sc, bn, h_out, hn_out):
    h = jnp.dot(x[...], wt[...], preferred_element_type=jnp.float32) + b[...]
    h_out[...] = h
    hn_out[...] = _ln(h, sc[...], bn[...])


def _tc_layer_body(h, hn, p0, p1, d0, d1, wlt, bl, wrt, sc, bn, h_out, hn_out):
    deg = jnp.maximum(d0[...][:, :1] + d1[...][:, :1], 1.0)
    agg = (p0[...] + p1[...]) / deg
    conv = (jnp.dot(agg, wlt[...], preferred_element_type=jnp.float32) + bl[...]
            + jnp.dot(hn[...], wrt[...], preferred_element_type=jnp.float32))
    hnew = jnp.maximum(h[...] + conv, 0.0)
    h_out[...] = hnew
    hn_out[...] = _ln(hnew, sc[...], bn[...])


def _tc_last_body(h, hn, p0, p1, d0, d1, wlt, bl, wrt, owt, ob, y_out):
    deg = jnp.maximum(d0[...][:, :1] + d1[...][:, :1], 1.0)
    agg = (p0[...] + p1[...]) / deg
    conv = (jnp.dot(agg, wlt[...], preferred_element_type=jnp.float32) + bl[...]
            + jnp.dot(hn[...], wrt[...], preferred_element_type=jnp.float32))
    hnew = jnp.maximum(h[...] + conv, 0.0)
    y_out[...] = jnp.dot(hnew, owt[...], preferred_element_type=jnp.float32) + ob[...]


def _row_spec(width=D):
    return pl.BlockSpec((R, width), lambda i: (i, 0))


def _full_spec(shape):
    return pl.BlockSpec(shape, lambda i: tuple(0 for _ in shape))


_tc_in = pl.pallas_call(
    _tc_in_body,
    grid=(GRID,),
    in_specs=[_row_spec(), _full_spec((D, D)), _full_spec((1, D)),
              _full_spec((1, D)), _full_spec((1, D))],
    out_specs=[_row_spec(), _row_spec()],
    out_shape=[jax.ShapeDtypeStruct((N, D), jnp.float32),
               jax.ShapeDtypeStruct((N, D), jnp.float32)],
)

_tc_layer = pl.pallas_call(
    _tc_layer_body,
    grid=(GRID,),
    in_specs=[_row_spec(), _row_spec(), _row_spec(), _row_spec(),
              _row_spec(16), _row_spec(16),
              _full_spec((D, D)), _full_spec((1, D)), _full_spec((D, D)),
              _full_spec((1, D)), _full_spec((1, D))],
    out_specs=[_row_spec(), _row_spec()],
    out_shape=[jax.ShapeDtypeStruct((N, D), jnp.float32),
               jax.ShapeDtypeStruct((N, D), jnp.float32)],
)

_tc_last = pl.pallas_call(
    _tc_last_body,
    grid=(GRID,),
    in_specs=[_row_spec(), _row_spec(), _row_spec(), _row_spec(),
              _row_spec(16), _row_spec(16),
              _full_spec((D, D)), _full_spec((1, D)), _full_spec((D, D)),
              _full_spec((D, OUT)), _full_spec((1, OUT))],
    out_specs=[_row_spec(OUT)],
    out_shape=[jax.ShapeDtypeStruct((N, OUT), jnp.float32)],
)


def kernel(x, edge_index, in_W, in_b, lin_l_W, lin_l_b, lin_r_W,
           ln_scale, ln_bias, out_W, out_b):
    src = edge_index[0]
    dst = edge_index[1]
    pad = E_PAD - E
    srcp = jnp.concatenate([src, jnp.zeros((pad,), jnp.int32)])
    dstp = jnp.concatenate([dst, jnp.full((pad,), N, jnp.int32)])
    zrows = jnp.zeros((ROWS_PER_TILE, D), jnp.float32)
    ones = jnp.ones((CHUNK, D), jnp.float32)

    h, hn = _tc_in(x, in_W.T, in_b[None], ln_scale[0][None], ln_bias[0][None])

    degparts = _make_sc_deg()(dstp, zrows, ones)
    parts = _make_sc_agg()(hn, srcp, dstp, zrows)
    d0 = degparts[0, :N, :16]
    d1 = degparts[1, :N, :16]

    for i in range(L):
        p0 = parts[0, :N]
        p1 = parts[1, :N]
        if i < L - 1:
            h, hn = _tc_layer(h, hn, p0, p1, d0, d1,
                              lin_l_W[i].T, lin_l_b[i][None], lin_r_W[i].T,
                              ln_scale[i + 1][None], ln_bias[i + 1][None])
            parts = _make_sc_agg()(hn, srcp, dstp, zrows)
        else:
            y = _tc_last(h, hn, p0, p1, d0, d1,
                         lin_l_W[i].T, lin_l_b[i][None], lin_r_W[i].T,
                         out_W.T, out_b[None])
    (y,) = y if isinstance(y, (list, tuple)) else (y,)
    return y
